# baseline (speedup 1.0000x reference)
import functools

import jax
import jax.numpy as jnp
from jax import lax
from jax.experimental import pallas as pl
from jax.experimental.pallas import tpu as pltpu
from jax.experimental.pallas import tpu_sc as plsc

_TR_BLOCK = 4096


def _split_transpose_tc(table_t, block_cols=_TR_BLOCK):
    d, v = table_t.shape
    half = block_cols // 2
    n_blocks = (v + block_cols - 1) // block_cols

    def body(t_ref, o_ref):
        tt = t_ref[...].T
        o_ref[:, 0:d] = tt[0:half]
        o_ref[:, d:2 * d] = tt[half:block_cols]

    return pl.pallas_call(
        body,
        grid=(n_blocks,),
        in_specs=[pl.BlockSpec((d, block_cols), lambda i: (0, i))],
        out_specs=pl.BlockSpec((half, 2 * d), lambda i: (i, 0)),
        out_shape=jax.ShapeDtypeStruct((n_blocks * half, 2 * d), jnp.float32),
    )(table_t)


def _bag_sum_sc(idx2d, table, hist, bags_per_chunk):
    info = plsc.get_sparse_core_info()
    nc, ns, lanes = info.num_cores, info.num_subcores, info.num_lanes
    nw = nc * ns
    n_chunks_total, chunk_idx = idx2d.shape
    assert chunk_idx == bags_per_chunk * hist
    _, d = table.shape
    n_bags = n_chunks_total * bags_per_chunk
    assert n_bags % (2 * nw) == 0
    bags_pw = n_bags // nw
    chunks_pw = n_chunks_total // nw
    assert chunks_pw % 2 == 0
    n_col = d // lanes

    mesh = plsc.VectorSubcoreMesh(core_axis_name="c", subcore_axis_name="s")

    @functools.partial(
        pl.kernel,
        out_type=jax.ShapeDtypeStruct((n_bags, d), jnp.float32),
        mesh=mesh,
        scratch_types=[
            pltpu.VMEM((chunks_pw, chunk_idx), jnp.int32),
            pltpu.VMEM((2, chunk_idx, d), jnp.float32),
            pltpu.VMEM((bags_pw, d), jnp.float32),
            pltpu.SemaphoreType.DMA,
            pltpu.SemaphoreType.DMA,
        ],
        compiler_params=pltpu.CompilerParams(use_tc_tiling_on_sc=False),
    )
    def k(idx_hbm, table_hbm, out_hbm, idx_v, rows_v, pooled_v, sem0, sem1):
        wid = lax.axis_index("s") * nc + lax.axis_index("c")
        pltpu.sync_copy(idx_hbm.at[pl.ds(wid * chunks_pw, chunks_pw), :], idx_v)

        def start(ci, buf, sem):
            pltpu.async_copy(table_hbm.at[idx_v.at[ci]], rows_v.at[buf], sem)

        def wait(buf, sem):
            pltpu.make_async_copy(
                table_hbm.at[idx_v.at[0]], rows_v.at[buf], sem
            ).wait()

        def compute(ci, buf):
            for b in range(bags_per_chunk):
                def row_body(r, accs):
                    base = b * hist + r
                    return tuple(
                        accs[c] + rows_v[buf, base, pl.ds(c * lanes, lanes)]
                        for c in range(n_col)
                    )
                accs = tuple(
                    jnp.zeros((lanes,), jnp.float32) for _ in range(n_col)
                )
                accs = lax.fori_loop(0, hist, row_body, accs)
                bag = ci * bags_per_chunk + b
                for c in range(n_col):
                    pooled_v[bag, pl.ds(c * lanes, lanes)] = accs[c]

        start(0, 0, sem0)

        def pair_body(ci2, _):
            ci = ci2 * 2
            start(ci + 1, 1, sem1)
            wait(0, sem0)
            compute(ci, 0)

            @pl.when(ci2 + 1 < chunks_pw // 2)
            def _():
                start(ci + 2, 0, sem0)

            wait(1, sem1)
            compute(ci + 1, 1)
            return 0

        lax.fori_loop(0, chunks_pw // 2, pair_body, 0)
        pltpu.sync_copy(
            pooled_v, out_hbm.at[pl.ds(wid * bags_pw, bags_pw), :]
        )

    return k(idx2d, table)


def _proj_tc(pooled, w, block_b=2048):
    b, d = pooled.shape
    o, _ = w.shape

    def body(p_ref, w_ref, o_ref):
        o_ref[...] = lax.dot_general(
            p_ref[...], w_ref[...],
            (((1,), (1,)), ((), ())),
            preferred_element_type=jnp.float32,
        )

    return pl.pallas_call(
        body,
        grid=(b // block_b,),
        in_specs=[
            pl.BlockSpec((block_b, d), lambda i: (i, 0)),
            pl.BlockSpec((o, d), lambda i: (0, 0)),
        ],
        out_specs=pl.BlockSpec((block_b, o), lambda i: (i, 0)),
        out_shape=jax.ShapeDtypeStruct((b, o), jnp.float32),
    )(pooled, w)


def kernel(input_, embed_weight, linear_weight):
    batch, hist = input_.shape
    nemb, d = embed_weight.shape
    bags_per_chunk = 2
    chunk_idx = bags_per_chunk * hist
    v = input_.astype(jnp.int32)
    half = _TR_BLOCK // 2
    j = (v & ~(_TR_BLOCK - 1)) | ((v & (half - 1)) << 1) | ((v >> 11) & 1)
    idx2d = j.reshape(batch // bags_per_chunk, chunk_idx)
    epairs = _split_transpose_tc(embed_weight.T)
    table = epairs.reshape(epairs.shape[0] * 2, d)
    pooled = _bag_sum_sc(idx2d, table, hist, bags_per_chunk)
    return _proj_tc(pooled, linear_weight)


# speedup vs baseline: 1.0589x; 1.0589x over previous
import functools

import jax
import jax.numpy as jnp
from jax import lax
from jax.experimental import pallas as pl
from jax.experimental.pallas import tpu as pltpu
from jax.experimental.pallas import tpu_sc as plsc

_TR_BLOCK = 4096


def _split_transpose_tc(table_t, block_cols=_TR_BLOCK):
    d, v = table_t.shape
    half = block_cols // 2
    n_blocks = (v + block_cols - 1) // block_cols
    eye = jnp.eye(d, dtype=jnp.bfloat16)

    def body(t_ref, e_ref, o_ref):
        tb = t_ref[...].astype(jnp.bfloat16)
        e = e_ref[...]
        o_ref[:, 0:d] = lax.dot_general(
            tb[:, 0:half], e, (((0,), (0,)), ((), ())),
            preferred_element_type=jnp.float32,
        )
        o_ref[:, d:2 * d] = lax.dot_general(
            tb[:, half:block_cols], e, (((0,), (0,)), ((), ())),
            preferred_element_type=jnp.float32,
        )

    return pl.pallas_call(
        body,
        grid=(n_blocks,),
        in_specs=[
            pl.BlockSpec((d, block_cols), lambda i: (0, i)),
            pl.BlockSpec((d, d), lambda i: (0, 0)),
        ],
        out_specs=pl.BlockSpec((half, 2 * d), lambda i: (i, 0)),
        out_shape=jax.ShapeDtypeStruct((n_blocks * half, 2 * d), jnp.float32),
    )(table_t, eye)


def _bag_sum_sc(idx2d, table, hist, bags_per_chunk):
    info = plsc.get_sparse_core_info()
    nc, ns, lanes = info.num_cores, info.num_subcores, info.num_lanes
    nw = nc * ns
    n_chunks_total, chunk_idx = idx2d.shape
    assert chunk_idx == bags_per_chunk * hist
    _, d = table.shape
    n_bags = n_chunks_total * bags_per_chunk
    assert n_bags % (2 * nw) == 0
    bags_pw = n_bags // nw
    chunks_pw = n_chunks_total // nw
    assert chunks_pw % 2 == 0
    n_col = d // lanes

    mesh = plsc.VectorSubcoreMesh(core_axis_name="c", subcore_axis_name="s")

    @functools.partial(
        pl.kernel,
        out_type=jax.ShapeDtypeStruct((n_bags, d), jnp.float32),
        mesh=mesh,
        scratch_types=[
            pltpu.VMEM((chunks_pw, chunk_idx), jnp.int32),
            pltpu.VMEM((2, chunk_idx, d), jnp.float32),
            pltpu.VMEM((bags_pw, d), jnp.float32),
            pltpu.SemaphoreType.DMA,
            pltpu.SemaphoreType.DMA,
        ],
        compiler_params=pltpu.CompilerParams(use_tc_tiling_on_sc=False),
    )
    def k(idx_hbm, table_hbm, out_hbm, idx_v, rows_v, pooled_v, sem0, sem1):
        wid = lax.axis_index("s") * nc + lax.axis_index("c")
        pltpu.sync_copy(idx_hbm.at[pl.ds(wid * chunks_pw, chunks_pw), :], idx_v)

        def start(ci, buf, sem):
            pltpu.async_copy(table_hbm.at[idx_v.at[ci]], rows_v.at[buf], sem)

        def wait(buf, sem):
            pltpu.make_async_copy(
                table_hbm.at[idx_v.at[0]], rows_v.at[buf], sem
            ).wait()

        def compute(ci, buf):
            for b in range(bags_per_chunk):
                def row_body(r, accs):
                    base = b * hist + r
                    return tuple(
                        accs[c] + rows_v[buf, base, pl.ds(c * lanes, lanes)]
                        for c in range(n_col)
                    )
                accs = tuple(
                    jnp.zeros((lanes,), jnp.float32) for _ in range(n_col)
                )
                accs = lax.fori_loop(0, hist, row_body, accs)
                bag = ci * bags_per_chunk + b
                for c in range(n_col):
                    pooled_v[bag, pl.ds(c * lanes, lanes)] = accs[c]

        start(0, 0, sem0)

        def pair_body(ci2, _):
            ci = ci2 * 2
            start(ci + 1, 1, sem1)
            wait(0, sem0)
            compute(ci, 0)

            @pl.when(ci2 + 1 < chunks_pw // 2)
            def _():
                start(ci + 2, 0, sem0)

            wait(1, sem1)
            compute(ci + 1, 1)
            return 0

        lax.fori_loop(0, chunks_pw // 2, pair_body, 0)
        pltpu.sync_copy(
            pooled_v, out_hbm.at[pl.ds(wid * bags_pw, bags_pw), :]
        )

    return k(idx2d, table)


def _proj_tc(pooled, w, block_b=2048):
    b, d = pooled.shape
    o, _ = w.shape

    def body(p_ref, w_ref, o_ref):
        o_ref[...] = lax.dot_general(
            p_ref[...], w_ref[...],
            (((1,), (1,)), ((), ())),
            preferred_element_type=jnp.float32,
        )

    return pl.pallas_call(
        body,
        grid=(b // block_b,),
        in_specs=[
            pl.BlockSpec((block_b, d), lambda i: (i, 0)),
            pl.BlockSpec((o, d), lambda i: (0, 0)),
        ],
        out_specs=pl.BlockSpec((block_b, o), lambda i: (i, 0)),
        out_shape=jax.ShapeDtypeStruct((b, o), jnp.float32),
    )(pooled, w)


def kernel(input_, embed_weight, linear_weight):
    batch, hist = input_.shape
    nemb, d = embed_weight.shape
    bags_per_chunk = 2
    chunk_idx = bags_per_chunk * hist
    v = input_.astype(jnp.int32)
    half = _TR_BLOCK // 2
    j = (v & ~(_TR_BLOCK - 1)) | ((v & (half - 1)) << 1) | ((v >> 11) & 1)
    idx2d = j.reshape(batch // bags_per_chunk, chunk_idx)
    epairs = _split_transpose_tc(embed_weight.T)
    table = epairs.reshape(epairs.shape[0] * 2, d)
    pooled = _bag_sum_sc(idx2d, table, hist, bags_per_chunk)
    return _proj_tc(pooled, linear_weight)


# speedup vs baseline: 1.2243x; 1.1562x over previous
import functools

import jax
import jax.numpy as jnp
from jax import lax
from jax.experimental import pallas as pl
from jax.experimental.pallas import tpu as pltpu
from jax.experimental.pallas import tpu_sc as plsc

_TR_BLOCK = 8192


def _split_transpose_tc(table_t, block_cols=_TR_BLOCK):
    d, v = table_t.shape
    half = block_cols // 2
    n_blocks = (v + block_cols - 1) // block_cols
    eye = jnp.eye(d, dtype=jnp.bfloat16)

    def body(t_ref, e_ref, o_ref):
        tb = t_ref[...].astype(jnp.bfloat16)
        e = e_ref[...]
        o_ref[:, 0:d] = lax.dot_general(
            tb[:, 0:half], e, (((0,), (0,)), ((), ())),
            preferred_element_type=jnp.float32,
        )
        o_ref[:, d:2 * d] = lax.dot_general(
            tb[:, half:block_cols], e, (((0,), (0,)), ((), ())),
            preferred_element_type=jnp.float32,
        )

    return pl.pallas_call(
        body,
        grid=(n_blocks,),
        in_specs=[
            pl.BlockSpec((d, block_cols), lambda i: (0, i)),
            pl.BlockSpec((d, d), lambda i: (0, 0)),
        ],
        out_specs=pl.BlockSpec((half, 2 * d), lambda i: (i, 0)),
        out_shape=jax.ShapeDtypeStruct((n_blocks * half, 2 * d), jnp.float32),
    )(table_t, eye)


def _bag_sum_sc(idx2d, table, hist, bags_per_chunk):
    info = plsc.get_sparse_core_info()
    nc, ns, lanes = info.num_cores, info.num_subcores, info.num_lanes
    nw = nc * ns
    n_chunks_total, chunk_idx = idx2d.shape
    assert chunk_idx == bags_per_chunk * hist
    _, d = table.shape
    n_bags = n_chunks_total * bags_per_chunk
    assert n_bags % (2 * nw) == 0
    bags_pw = n_bags // nw
    chunks_pw = n_chunks_total // nw
    assert chunks_pw % 2 == 0
    n_col = d // lanes

    mesh = plsc.VectorSubcoreMesh(core_axis_name="c", subcore_axis_name="s")

    @functools.partial(
        pl.kernel,
        out_type=jax.ShapeDtypeStruct((n_bags, d), jnp.float32),
        mesh=mesh,
        scratch_types=[
            pltpu.VMEM((chunks_pw, chunk_idx), jnp.int32),
            pltpu.VMEM((2, chunk_idx, d), jnp.float32),
            pltpu.VMEM((bags_pw, d), jnp.float32),
            pltpu.SemaphoreType.DMA,
            pltpu.SemaphoreType.DMA,
        ],
        compiler_params=pltpu.CompilerParams(use_tc_tiling_on_sc=False),
    )
    def k(idx_hbm, table_hbm, out_hbm, idx_v, rows_v, pooled_v, sem0, sem1):
        wid = lax.axis_index("s") * nc + lax.axis_index("c")
        pltpu.sync_copy(idx_hbm.at[pl.ds(wid * chunks_pw, chunks_pw), :], idx_v)

        def start(ci, buf, sem):
            pltpu.async_copy(table_hbm.at[idx_v.at[ci]], rows_v.at[buf], sem)

        def wait(buf, sem):
            pltpu.make_async_copy(
                table_hbm.at[idx_v.at[0]], rows_v.at[buf], sem
            ).wait()

        def compute(ci, buf):
            for b in range(bags_per_chunk):
                def row_body(r, accs):
                    base = b * hist + r
                    return tuple(
                        accs[c] + rows_v[buf, base, pl.ds(c * lanes, lanes)]
                        for c in range(n_col)
                    )
                accs = tuple(
                    jnp.zeros((lanes,), jnp.float32) for _ in range(n_col)
                )
                accs = lax.fori_loop(0, hist, row_body, accs)
                bag = ci * bags_per_chunk + b
                for c in range(n_col):
                    pooled_v[bag, pl.ds(c * lanes, lanes)] = accs[c]

        start(0, 0, sem0)

        def pair_body(ci2, _):
            ci = ci2 * 2
            start(ci + 1, 1, sem1)
            wait(0, sem0)
            compute(ci, 0)

            @pl.when(ci2 + 1 < chunks_pw // 2)
            def _():
                start(ci + 2, 0, sem0)

            wait(1, sem1)
            compute(ci + 1, 1)
            return 0

        lax.fori_loop(0, chunks_pw // 2, pair_body, 0)
        pltpu.sync_copy(
            pooled_v, out_hbm.at[pl.ds(wid * bags_pw, bags_pw), :]
        )

    return k(idx2d, table)


def _proj_tc(pooled, w, block_b=2048):
    b, d = pooled.shape
    o, _ = w.shape

    def body(p_ref, w_ref, o_ref):
        o_ref[...] = lax.dot_general(
            p_ref[...], w_ref[...],
            (((1,), (1,)), ((), ())),
            preferred_element_type=jnp.float32,
        )

    return pl.pallas_call(
        body,
        grid=(b // block_b,),
        in_specs=[
            pl.BlockSpec((block_b, d), lambda i: (i, 0)),
            pl.BlockSpec((o, d), lambda i: (0, 0)),
        ],
        out_specs=pl.BlockSpec((block_b, o), lambda i: (i, 0)),
        out_shape=jax.ShapeDtypeStruct((b, o), jnp.float32),
    )(pooled, w)


def kernel(input_, embed_weight, linear_weight):
    batch, hist = input_.shape
    nemb, d = embed_weight.shape
    bags_per_chunk = 2
    chunk_idx = bags_per_chunk * hist
    v = input_.astype(jnp.int32)
    half = _TR_BLOCK // 2
    shift = half.bit_length() - 1
    j = (v & ~(_TR_BLOCK - 1)) | ((v & (half - 1)) << 1) | ((v >> shift) & 1)
    idx2d = j.reshape(batch // bags_per_chunk, chunk_idx)
    epairs = _split_transpose_tc(embed_weight.T)
    table = epairs.reshape(epairs.shape[0] * 2, d)
    pooled = _bag_sum_sc(idx2d, table, hist, bags_per_chunk)
    return _proj_tc(pooled, linear_weight)
